# Initial kernel scaffold; baseline (speedup 1.0000x reference)
import functools

import jax
import jax.numpy as jnp
from jax import lax
from jax.experimental import pallas as pl
from jax.experimental.pallas import tpu as pltpu
from jax.experimental.pallas import tpu_sc as plsc

N = 10000
E = 320000
D = 128
H = 128
R = 8
L = 2
T = 16
M = 1024

NC = 2
NS = 16
NW = NC * NS
EPT = E // NW
EB = 80
NB = EPT // EB
RN = R * N
RNP = 81920
RPT = RNP // NS
NPT = N // NS

_f32 = jnp.float32
_i32 = jnp.int32

_MESH = plsc.VectorSubcoreMesh(core_axis_name="c", subcore_axis_name="s")


def _wid():
    return lax.axis_index("s") * NC + lax.axis_index("c")


@functools.partial(
    pl.kernel,
    out_type=jax.ShapeDtypeStruct((NC, RNP, 16), _f32),
    mesh=_MESH,
    scratch_types=[
        pltpu.VMEM((EB,), _i32),
        pltpu.VMEM((EB,), _i32),
        pltpu.VMEM((EB,), _i32),
        pltpu.VMEM((EB, 16), _f32),
        pltpu.VMEM((512, 16), _f32),
        pltpu.VMEM_SHARED((RNP, 16), _f32),
    ],
)
def _count_kernel(et_hbm, dst_hbm, cnt_hbm, et_v, dst_v, cidx_v, ones_v, zero_v, acc_sh):
    cid = lax.axis_index("c")
    sid = lax.axis_index("s")
    wid = sid * NC + cid

    def fill_zero(i, carry):
        zero_v[i, :] = jnp.zeros((16,), _f32)
        return carry

    lax.fori_loop(0, 512, fill_zero, 0)

    def fill_ones(i, carry):
        ones_v[i, :] = jnp.ones((16,), _f32)
        return carry

    lax.fori_loop(0, EB, fill_ones, 0)

    row0 = sid * RPT

    def zchunk(k, carry):
        pltpu.sync_copy(zero_v, acc_sh.at[pl.ds(row0 + k * 512, 512)])
        return carry

    lax.fori_loop(0, RPT // 512, zchunk, 0)
    plsc.subcore_barrier()

    ebase = wid * EPT

    def ebatch(b, carry):
        off = ebase + b * EB
        pltpu.sync_copy(et_hbm.at[pl.ds(off, EB)], et_v)
        pltpu.sync_copy(dst_hbm.at[pl.ds(off, EB)], dst_v)
        for g in range(EB // 16):
            s = pl.ds(g * 16, 16)
            cidx_v[s] = et_v[s] * N + dst_v[s]
        pltpu.sync_copy(ones_v, acc_sh.at[cidx_v], add=True)
        return carry

    lax.fori_loop(0, NB, ebatch, 0)
    plsc.subcore_barrier()
    pltpu.sync_copy(acc_sh.at[pl.ds(row0, RPT)], cnt_hbm.at[cid, pl.ds(row0, RPT)])


@functools.partial(
    pl.kernel,
    out_type=jax.ShapeDtypeStruct((E,), _f32),
    mesh=_MESH,
    scratch_types=[
        pltpu.VMEM((512, 16), _f32),
        pltpu.VMEM((512, 16), _f32),
        pltpu.VMEM((512,), _f32),
        pltpu.VMEM((RNP,), _f32),
        pltpu.VMEM((EB,), _i32),
        pltpu.VMEM((EB,), _i32),
        pltpu.VMEM((EB,), _f32),
        pltpu.VMEM_SHARED((RNP,), _f32),
    ],
)
def _w_kernel(cnt_hbm, et_hbm, dst_hbm, w_hbm,
              p0_v, p1_v, rd_v, rdfull_v, et_v, dst_v, w_v, rdeg_sh):
    cid = lax.axis_index("c")
    sid = lax.axis_index("s")
    wid = sid * NC + cid
    tbase = sid * RPT
    z16 = jnp.zeros((16,), _i32)

    def chunk(k, carry):
        r0 = tbase + k * 512
        pltpu.sync_copy(cnt_hbm.at[0, pl.ds(r0, 512)], p0_v)
        pltpu.sync_copy(cnt_hbm.at[1, pl.ds(r0, 512)], p1_v)

        def grp(g, c2):
            r16 = lax.broadcasted_iota(_i32, (16,), 0) + g * 16
            v0 = plsc.load_gather(p0_v, [r16, z16])
            v1 = plsc.load_gather(p1_v, [r16, z16])
            rd_v[pl.ds(g * 16, 16)] = 1.0 / jnp.maximum(v0 + v1, 1.0)
            return c2

        lax.fori_loop(0, 512 // 16, grp, 0)
        pltpu.sync_copy(rd_v, rdeg_sh.at[pl.ds(r0, 512)])
        return carry

    lax.fori_loop(0, RPT // 512, chunk, 0)
    plsc.subcore_barrier()
    pltpu.sync_copy(rdeg_sh, rdfull_v)

    ebase = wid * EPT

    def ebatch(b, carry):
        off = ebase + b * EB
        pltpu.sync_copy(et_hbm.at[pl.ds(off, EB)], et_v)
        pltpu.sync_copy(dst_hbm.at[pl.ds(off, EB)], dst_v)
        for g in range(EB // 16):
            s = pl.ds(g * 16, 16)
            w_v[s] = plsc.load_gather(rdfull_v, [et_v[s] * N + dst_v[s]])
        pltpu.sync_copy(w_v, w_hbm.at[pl.ds(off, EB)])
        return carry

    lax.fori_loop(0, NB, ebatch, 0)


@functools.partial(
    pl.kernel,
    out_type=jax.ShapeDtypeStruct((NC, N, H), _f32),
    mesh=_MESH,
    scratch_types=[
        pltpu.VMEM((EB,), _i32),
        pltpu.VMEM((EB,), _i32),
        pltpu.VMEM((EB,), _i32),
        pltpu.VMEM((EB,), _i32),
        pltpu.VMEM((EB,), _f32),
        pltpu.VMEM((EB, H), _f32),
        pltpu.VMEM((125, H), _f32),
        pltpu.SemaphoreType.DMA,
        pltpu.VMEM_SHARED((N, H), _f32),
    ],
)
def _scatter_kernel(hr_hbm, src_hbm, et_hbm, dst_hbm, w_hbm, acc_hbm,
                    src_v, et_v, dst_v, gidx_v, w_v, rows_v, zero_v, sem, acc_sh):
    cid = lax.axis_index("c")
    sid = lax.axis_index("s")
    wid = sid * NC + cid

    def fill_zero(i, carry):
        for c in range(H // 16):
            zero_v[i, pl.ds(c * 16, 16)] = jnp.zeros((16,), _f32)
        return carry

    lax.fori_loop(0, 125, fill_zero, 0)

    row0 = sid * NPT

    def zchunk(k, carry):
        pltpu.sync_copy(zero_v, acc_sh.at[pl.ds(row0 + k * 125, 125)])
        return carry

    lax.fori_loop(0, NPT // 125, zchunk, 0)
    plsc.subcore_barrier()

    ebase = wid * EPT

    def ebatch(b, carry):
        off = ebase + b * EB
        pltpu.sync_copy(src_hbm.at[pl.ds(off, EB)], src_v)
        pltpu.sync_copy(et_hbm.at[pl.ds(off, EB)], et_v)
        pltpu.sync_copy(dst_hbm.at[pl.ds(off, EB)], dst_v)
        pltpu.sync_copy(w_hbm.at[pl.ds(off, EB)], w_v)
        for g in range(EB // 16):
            s = pl.ds(g * 16, 16)
            gidx_v[s] = et_v[s] * N + src_v[s]
        pltpu.async_copy(hr_hbm.at[gidx_v], rows_v, sem).wait()

        def scale_grp(g, c2):
            w16 = w_v[pl.ds(g * 16, 16)]
            for l in range(16):
                wb = jnp.broadcast_to(w16[l], (16,))
                e = g * 16 + l
                for c in range(H // 16):
                    s = pl.ds(c * 16, 16)
                    rows_v[e, s] = rows_v[e, s] * wb
            return c2

        lax.fori_loop(0, EB // 16, scale_grp, 0)
        pltpu.sync_copy(rows_v, acc_sh.at[dst_v], add=True)
        return carry

    lax.fori_loop(0, NB, ebatch, 0)
    plsc.subcore_barrier()
    pltpu.sync_copy(acc_sh.at[pl.ds(row0, NPT)], acc_hbm.at[cid, pl.ds(row0, NPT)])


GB = M + 256
GPT = GB // NW


@functools.partial(
    pl.kernel,
    out_type=jax.ShapeDtypeStruct((GB, H), _f32),
    mesh=_MESH,
    scratch_types=[
        pltpu.VMEM((GPT,), _i32),
        pltpu.VMEM((GPT, H), _f32),
        pltpu.SemaphoreType.DMA,
    ],
)
def _final_gather_kernel(h_hbm, idx_hbm, out_hbm, idx_v, rows_v, sem):
    wid = _wid()
    base = wid * GPT
    pltpu.sync_copy(idx_hbm.at[pl.ds(base, GPT)], idx_v)
    pltpu.async_copy(h_hbm.at[idx_v], rows_v, sem).wait()
    pltpu.sync_copy(rows_v, out_hbm.at[pl.ds(base, GPT)])


BN = 1000
NG = N // BN


def _embed_body(nf_ref, ids_ref, fw_ref, fb_ref, te_ref, out_ref):
    ids = ids_ref[0, 0, :]
    oh = (ids[:, None] == lax.broadcasted_iota(_i32, (1, T), 1)).astype(_f32)
    out_ref[...] = (
        jnp.dot(nf_ref[...], fw_ref[...], preferred_element_type=_f32)
        + jnp.dot(oh, te_ref[...], preferred_element_type=_f32)
        + fb_ref[...]
    )


def _embed_call(nf, ids3, fw, fb, te):
    return pl.pallas_call(
        _embed_body,
        grid=(NG,),
        in_specs=[
            pl.BlockSpec((BN, D), lambda i: (i, 0)),
            pl.BlockSpec((1, 1, BN), lambda i: (i, 0, 0)),
            pl.BlockSpec((D, H), lambda i: (0, 0)),
            pl.BlockSpec((1, H), lambda i: (0, 0)),
            pl.BlockSpec((T, H), lambda i: (0, 0)),
        ],
        out_specs=pl.BlockSpec((BN, H), lambda i: (i, 0)),
        out_shape=jax.ShapeDtypeStruct((N, H), _f32),
    )(nf, ids3, fw, fb, te)


def _hr_body(h_ref, rw_ref, out_ref):
    out_ref[...] = jnp.dot(h_ref[...], rw_ref[0], preferred_element_type=_f32)[None]


def _hr_call(h, rw):
    return pl.pallas_call(
        _hr_body,
        grid=(NG, R),
        in_specs=[
            pl.BlockSpec((BN, H), lambda i, r: (i, 0)),
            pl.BlockSpec((1, H, H), lambda i, r: (r, 0, 0)),
        ],
        out_specs=pl.BlockSpec((1, BN, H), lambda i, r: (r, i, 0)),
        out_shape=jax.ShapeDtypeStruct((R, N, H), _f32),
    )(h, rw)


def _combine_body(h_ref, sw_ref, sb_ref, a0_ref, a1_ref, g_ref, b_ref, out_ref):
    o = jnp.dot(h_ref[...], sw_ref[...], preferred_element_type=_f32) + sb_ref[...]
    o = o + a0_ref[...] + a1_ref[...]
    o = jnp.maximum(o, 0.0)
    mu = jnp.mean(o, axis=-1, keepdims=True)
    d = o - mu
    var = jnp.mean(d * d, axis=-1, keepdims=True)
    out_ref[...] = d * lax.rsqrt(var + 1e-5) * g_ref[...] + b_ref[...]


def _combine_call(h, sw, sb, a0, a1, g, b):
    return pl.pallas_call(
        _combine_body,
        grid=(NG,),
        in_specs=[
            pl.BlockSpec((BN, H), lambda i: (i, 0)),
            pl.BlockSpec((H, H), lambda i: (0, 0)),
            pl.BlockSpec((1, H), lambda i: (0, 0)),
            pl.BlockSpec((BN, H), lambda i: (i, 0)),
            pl.BlockSpec((BN, H), lambda i: (i, 0)),
            pl.BlockSpec((1, H), lambda i: (0, 0)),
            pl.BlockSpec((1, H), lambda i: (0, 0)),
        ],
        out_specs=pl.BlockSpec((BN, H), lambda i: (i, 0)),
        out_shape=jax.ShapeDtypeStruct((N, H), _f32),
    )(h, sw, sb, a0, a1, g, b)


def kernel(node_features, node_type_ids, edge_index, edge_type, target_node_idx,
           memory_node_indices, type_emb, feat_W, feat_b, self_W, self_b, rel_W,
           ln_g, ln_b):
    src = edge_index[0]
    dst = edge_index[1]
    et = edge_type
    ids3 = node_type_ids.reshape(NG, 1, BN)

    h = _embed_call(node_features, ids3, feat_W, feat_b.reshape(1, H), type_emb)
    cnt = _count_kernel(et, dst)
    w = _w_kernel(cnt, et, dst)

    for l in range(L):
        hr = _hr_call(h, rel_W[l]).reshape(RN, H)
        acc = _scatter_kernel(hr, src, et, dst, w)
        h = _combine_call(h, self_W[l], self_b[l].reshape(1, H),
                          acc[0], acc[1],
                          ln_g[l].reshape(1, H), ln_b[l].reshape(1, H))

    tgt = jnp.full((GB - M,), target_node_idx, _i32)
    gidx = jnp.concatenate([memory_node_indices.astype(_i32), tgt])
    rows = _final_gather_kernel(h, gidx)
    return rows[M], rows[:M]


# speedup vs baseline: 3.3281x; 3.3281x over previous
import functools

import jax
import jax.numpy as jnp
from jax import lax
from jax.experimental import pallas as pl
from jax.experimental.pallas import tpu as pltpu
from jax.experimental.pallas import tpu_sc as plsc

N = 10000
E = 320000
D = 128
H = 128
R = 8
L = 2
T = 16
M = 1024

NC = 2
NS = 16
NW = NC * NS
EB = 80
EPW = E // NW
EPS = E // NS
RN = R * N
HN = N // NC
DACC = 5120
DDEAD = DACC - 1
DPT = DACC // NS
ACC = 10240
APT = ACC // NS
WBT = 624

_f32 = jnp.float32
_i32 = jnp.int32

_MESH = plsc.VectorSubcoreMesh(core_axis_name="c", subcore_axis_name="s")


@functools.partial(
    pl.kernel,
    out_type=jax.ShapeDtypeStruct((NC, DACC, H), _f32),
    mesh=_MESH,
    scratch_types=[
        pltpu.VMEM((EB,), _i32),
        pltpu.VMEM((EB,), _i32),
        pltpu.VMEM((EB,), _i32),
        pltpu.VMEM((EB, H), _f32),
        pltpu.VMEM((160, H), _f32),
        pltpu.SemaphoreType.DMA,
        pltpu.VMEM_SHARED((DACC, H), _f32),
    ],
)
def _deg_kernel(onesrel_hbm, et_hbm, dst_hbm, deg_hbm,
                et_v, dst_v, lidx_v, rows_v, zero_v, sem, acc_sh):
    cid = lax.axis_index("c")
    sid = lax.axis_index("s")
    z16f = jnp.zeros((16,), _f32)

    def fill_zero(i, carry):
        for c in range(H // 16):
            zero_v[i, pl.ds(c * 16, 16)] = z16f
        return carry

    lax.fori_loop(0, 160, fill_zero, 0)

    row0 = sid * DPT

    def zchunk(k, carry):
        pltpu.sync_copy(zero_v, acc_sh.at[pl.ds(row0 + k * 160, 160)])
        return carry

    lax.fori_loop(0, DPT // 160, zchunk, 0)
    plsc.subcore_barrier()

    lo = cid * HN
    ebase = sid * EPS

    def ebatch(b, carry):
        off = ebase + b * EB
        pltpu.sync_copy(et_hbm.at[pl.ds(off, EB)], et_v)
        pltpu.sync_copy(dst_hbm.at[pl.ds(off, EB)], dst_v)
        for g in range(EB // 16):
            s = pl.ds(g * 16, 16)
            local = dst_v[s] - lo
            own = jnp.logical_and(local >= 0, local < HN)
            lidx_v[s] = jnp.where(own, local, DDEAD)
        pltpu.async_copy(onesrel_hbm.at[et_v], rows_v, sem).wait()
        pltpu.sync_copy(rows_v, acc_sh.at[lidx_v], add=True)
        return carry

    lax.fori_loop(0, EPS // EB, ebatch, 0)
    plsc.subcore_barrier()
    pltpu.sync_copy(acc_sh.at[pl.ds(row0, DPT)],
                    deg_hbm.at[cid, pl.ds(row0, DPT)])


@functools.partial(
    pl.kernel,
    out_type=jax.ShapeDtypeStruct((E, 16), _f32),
    mesh=_MESH,
    scratch_types=[
        pltpu.VMEM((EB,), _i32),
        pltpu.VMEM((EB,), _i32),
        pltpu.VMEM((EB,), _i32),
        pltpu.VMEM((EB, H), _f32),
        pltpu.VMEM((EB, 16), _f32),
        pltpu.SemaphoreType.DMA,
    ],
)
def _wprep_kernel(rdeg_hbm, et_hbm, dst_hbm, w_hbm,
                  et_v, dst_v, widx_v, rows_v, w16_v, sem):
    cid = lax.axis_index("c")
    sid = lax.axis_index("s")
    wid = sid * NC + cid
    ebase = wid * EPW

    def ebatch(b, carry):
        off = ebase + b * EB
        pltpu.sync_copy(et_hbm.at[pl.ds(off, EB)], et_v)
        pltpu.sync_copy(dst_hbm.at[pl.ds(off, EB)], dst_v)
        for g in range(EB // 16):
            s = pl.ds(g * 16, 16)
            widx_v[s] = dst_v[s] * R + et_v[s]
        pltpu.async_copy(rdeg_hbm.at[widx_v], rows_v, sem).wait()

        def squeeze_grp(g, c2):
            for l in range(16):
                e = g * 16 + l
                w16_v[e, :] = rows_v[e, pl.ds(0, 16)]
            return c2

        lax.fori_loop(0, EB // 16, squeeze_grp, 0)
        pltpu.sync_copy(w16_v, w_hbm.at[pl.ds(off, EB)])
        return carry

    lax.fori_loop(0, EPW // EB, ebatch, 0)


@functools.partial(
    pl.kernel,
    out_type=jax.ShapeDtypeStruct((NC, N, H), _f32),
    mesh=_MESH,
    scratch_types=[
        pltpu.VMEM((EB,), _i32),
        pltpu.VMEM((EB,), _i32),
        pltpu.VMEM((EB,), _i32),
        pltpu.VMEM((EB,), _i32),
        pltpu.VMEM((EB, H), _f32),
        pltpu.VMEM((EB, 16), _f32),
        pltpu.VMEM((128, H), _f32),
        pltpu.SemaphoreType.DMA,
        pltpu.VMEM_SHARED((ACC, H), _f32),
    ],
)
def _scatter_kernel(hr_hbm, w_hbm, src_hbm, et_hbm, dst_hbm, acc_hbm,
                    src_v, et_v, dst_v, gidx_v, rows_v, wrows_v,
                    zero_v, sem, acc_sh):
    cid = lax.axis_index("c")
    sid = lax.axis_index("s")
    wid = sid * NC + cid
    z16f = jnp.zeros((16,), _f32)

    def fill_zero(i, carry):
        for c in range(H // 16):
            zero_v[i, pl.ds(c * 16, 16)] = z16f
        return carry

    lax.fori_loop(0, 128, fill_zero, 0)

    zrow0 = sid * APT

    def zchunk(k, carry):
        pltpu.sync_copy(zero_v, acc_sh.at[pl.ds(zrow0 + k * 128, 128)])
        return carry

    lax.fori_loop(0, APT // 128, zchunk, 0)
    plsc.subcore_barrier()

    ebase = wid * EPW

    def ebatch(b, carry):
        off = ebase + b * EB
        pltpu.sync_copy(src_hbm.at[pl.ds(off, EB)], src_v)
        pltpu.sync_copy(et_hbm.at[pl.ds(off, EB)], et_v)
        pltpu.sync_copy(dst_hbm.at[pl.ds(off, EB)], dst_v)
        pltpu.sync_copy(w_hbm.at[pl.ds(off, EB)], wrows_v)
        for g in range(EB // 16):
            s = pl.ds(g * 16, 16)
            gidx_v[s] = et_v[s] * N + src_v[s]
        pltpu.async_copy(hr_hbm.at[gidx_v], rows_v, sem).wait()

        def scale_grp(g, c2):
            for l in range(16):
                e = g * 16 + l
                wrow = wrows_v[e, :]
                for c in range(H // 16):
                    s = pl.ds(c * 16, 16)
                    rows_v[e, s] = rows_v[e, s] * wrow
            return c2

        lax.fori_loop(0, EB // 16, scale_grp, 0)
        pltpu.sync_copy(rows_v, acc_sh.at[dst_v], add=True)
        return carry

    lax.fori_loop(0, EPW // EB, ebatch, 0)
    plsc.subcore_barrier()
    pltpu.sync_copy(acc_sh.at[pl.ds(sid * WBT, WBT)],
                    acc_hbm.at[cid, pl.ds(sid * WBT, WBT)])

    @pl.when(sid == 0)
    def _tail():
        pltpu.sync_copy(acc_sh.at[pl.ds(NS * WBT, N - NS * WBT)],
                        acc_hbm.at[cid, pl.ds(NS * WBT, N - NS * WBT)])


GB = M + 256
GPT = GB // NW


@functools.partial(
    pl.kernel,
    out_type=jax.ShapeDtypeStruct((GB, H), _f32),
    mesh=_MESH,
    scratch_types=[
        pltpu.VMEM((GPT,), _i32),
        pltpu.VMEM((GPT, H), _f32),
        pltpu.SemaphoreType.DMA,
    ],
)
def _final_gather_kernel(h_hbm, idx_hbm, out_hbm, idx_v, rows_v, sem):
    wid = lax.axis_index("s") * NC + lax.axis_index("c")
    base = wid * GPT
    pltpu.sync_copy(idx_hbm.at[pl.ds(base, GPT)], idx_v)
    pltpu.async_copy(h_hbm.at[idx_v], rows_v, sem).wait()
    pltpu.sync_copy(rows_v, out_hbm.at[pl.ds(base, GPT)])


BN = 1000
NG = N // BN


def _embed_body(nf_ref, ids_ref, fw_ref, fb_ref, te_ref, out_ref):
    ids = ids_ref[0, 0, :]
    oh = (ids[:, None] == lax.broadcasted_iota(_i32, (1, T), 1)).astype(_f32)
    out_ref[...] = (
        jnp.dot(nf_ref[...], fw_ref[...], preferred_element_type=_f32)
        + jnp.dot(oh, te_ref[...], preferred_element_type=_f32)
        + fb_ref[...]
    )


def _embed_call(nf, ids3, fw, fb, te):
    return pl.pallas_call(
        _embed_body,
        grid=(NG,),
        in_specs=[
            pl.BlockSpec((BN, D), lambda i: (i, 0)),
            pl.BlockSpec((1, 1, BN), lambda i: (i, 0, 0)),
            pl.BlockSpec((D, H), lambda i: (0, 0)),
            pl.BlockSpec((1, H), lambda i: (0, 0)),
            pl.BlockSpec((T, H), lambda i: (0, 0)),
        ],
        out_specs=pl.BlockSpec((BN, H), lambda i: (i, 0)),
        out_shape=jax.ShapeDtypeStruct((N, H), _f32),
    )(nf, ids3, fw, fb, te)


def _expand_body(deg_ref, out_ref):
    d = deg_ref[0]
    msel = (lax.broadcasted_iota(_i32, (H, R), 0) // 16
            == lax.broadcasted_iota(_i32, (H, R), 1)).astype(_f32) * (1.0 / 16.0)
    deg8 = jnp.dot(d, msel, preferred_element_type=_f32)
    rd8 = 1.0 / jnp.maximum(deg8, 1.0)
    out_ref[...] = jnp.broadcast_to(rd8[:, :, None], (BN, R, H))


def _expand_call(deg):
    return pl.pallas_call(
        _expand_body,
        grid=(NG,),
        in_specs=[
            pl.BlockSpec((1, BN, H), lambda i: (i // (NG // NC), i % (NG // NC), 0)),
        ],
        out_specs=pl.BlockSpec((BN, R, H), lambda i: (i, 0, 0)),
        out_shape=jax.ShapeDtypeStruct((N, R, H), _f32),
    )(deg)


def _hr_body(h_ref, rw_ref, out_ref):
    out_ref[...] = jnp.dot(h_ref[...], rw_ref[0], preferred_element_type=_f32)[None]


def _hr_call(h, rw):
    return pl.pallas_call(
        _hr_body,
        grid=(NG, R),
        in_specs=[
            pl.BlockSpec((BN, H), lambda i, r: (i, 0)),
            pl.BlockSpec((1, H, H), lambda i, r: (r, 0, 0)),
        ],
        out_specs=pl.BlockSpec((1, BN, H), lambda i, r: (r, i, 0)),
        out_shape=jax.ShapeDtypeStruct((R, N, H), _f32),
    )(h, rw)


def _combine_body(h_ref, sw_ref, sb_ref, a0_ref, a1_ref, g_ref, b_ref, out_ref):
    o = jnp.dot(h_ref[...], sw_ref[...], preferred_element_type=_f32) + sb_ref[...]
    o = o + a0_ref[...] + a1_ref[...]
    o = jnp.maximum(o, 0.0)
    mu = jnp.mean(o, axis=-1, keepdims=True)
    d = o - mu
    var = jnp.mean(d * d, axis=-1, keepdims=True)
    out_ref[...] = d * lax.rsqrt(var + 1e-5) * g_ref[...] + b_ref[...]


def _combine_call(h, sw, sb, a0, a1, g, b):
    return pl.pallas_call(
        _combine_body,
        grid=(NG,),
        in_specs=[
            pl.BlockSpec((BN, H), lambda i: (i, 0)),
            pl.BlockSpec((H, H), lambda i: (0, 0)),
            pl.BlockSpec((1, H), lambda i: (0, 0)),
            pl.BlockSpec((BN, H), lambda i: (i, 0)),
            pl.BlockSpec((BN, H), lambda i: (i, 0)),
            pl.BlockSpec((1, H), lambda i: (0, 0)),
            pl.BlockSpec((1, H), lambda i: (0, 0)),
        ],
        out_specs=pl.BlockSpec((BN, H), lambda i: (i, 0)),
        out_shape=jax.ShapeDtypeStruct((N, H), _f32),
    )(h, sw, sb, a0, a1, g, b)


def kernel(node_features, node_type_ids, edge_index, edge_type, target_node_idx,
           memory_node_indices, type_emb, feat_W, feat_b, self_W, self_b, rel_W,
           ln_g, ln_b):
    src = edge_index[0]
    dst = edge_index[1]
    et = edge_type
    ids3 = node_type_ids.reshape(NG, 1, BN)
    onesrel = (lax.broadcasted_iota(_i32, (R, H), 1) // 16
               == lax.broadcasted_iota(_i32, (R, H), 0)).astype(_f32)

    h = _embed_call(node_features, ids3, feat_W, feat_b.reshape(1, H), type_emb)
    deg = _deg_kernel(onesrel, et, dst)
    rdeg = _expand_call(deg).reshape(N * R, H)
    w16 = _wprep_kernel(rdeg, et, dst)

    def layer_step(hc, ws):
        rw, sw, sb, g, b = ws
        hr = _hr_call(hc, rw).reshape(RN, H)
        acc = _scatter_kernel(hr, w16, src, et, dst)
        hn = _combine_call(hc, sw, sb.reshape(1, H), acc[0], acc[1],
                           g.reshape(1, H), b.reshape(1, H))
        return hn, None

    h, _ = lax.scan(layer_step, h, (rel_W, self_W, self_b, ln_g, ln_b))

    tgt = jnp.full((GB - M,), target_node_idx, _i32)
    gidx = jnp.concatenate([memory_node_indices.astype(_i32), tgt])
    rows = _final_gather_kernel(h, gidx)
    return rows[M], rows[:M]
